# baseline (speedup 1.0000x reference)
import functools

import jax
import jax.numpy as jnp
import numpy as np
from jax import lax
from jax.experimental import pallas as pl
from jax.experimental.pallas import tpu as pltpu
from jax.experimental.pallas import tpu_sc as plsc

N = 10000
E = 320000
IN_F = 128
OUT_F = 128
K = 4
DIM = 4

NC = 2
NS = 16
NW = NC * NS
EPW = E // NW
B = 32
TAIL = EPW % B
NCHUNK = EPW // B
UNROLL = 4
T_ITERS = NCHUNK // UNROLL
LANES = 16
OB = UNROLL * B

BN = 400
BE = E // (N // BN)
BN2 = 400

ROWS8 = 632
ROWS_LAST = N - ROWS8 * (NS - 1)

_PERM = np.empty(OUT_F, np.int32)
for _g in range(OUT_F // 32):
    for _l in range(LANES):
        _PERM[32 * _g + _l] = 32 * _g + 2 * _l
        _PERM[32 * _g + LANES + _l] = 32 * _g + 2 * _l + 1
_INV_PERM = np.empty(OUT_F, np.int32)
_INV_PERM[_PERM] = np.arange(OUT_F, dtype=np.int32)


def _pre_kernel(mu_ref, isig_ref, feat_ref, w_ref, pt_ref, h4_ref, wt_ref):
    hfull = lax.dot_general(
        feat_ref[...], w_ref[...], (((1,), (1,)), ((), ())),
        preferred_element_type=jnp.float32)
    for k in range(K):
        h4_ref[k] = hfull[:, k * OUT_F:(k + 1) * OUT_F].astype(jnp.bfloat16)
    for k in range(K):
        acc = None
        for d in range(DIM):
            p = pt_ref[d:d + 1, :]
            diff = p - mu_ref[k, d]
            t = (isig_ref[k, d] * isig_ref[k, d]) * diff * diff
            acc = t if acc is None else acc + t
        wt_ref[k:k + 1, :] = jnp.exp(-0.5 * acc)


def _tc_pre(feat, W, pseudo_t, mu, inv_sigma):
    return pl.pallas_call(
        _pre_kernel,
        grid=(N // BN,),
        in_specs=[
            pl.BlockSpec(memory_space=pltpu.SMEM),
            pl.BlockSpec(memory_space=pltpu.SMEM),
            pl.BlockSpec((BN, IN_F), lambda i: (i, 0)),
            pl.BlockSpec((K * OUT_F, IN_F), lambda i: (0, 0)),
            pl.BlockSpec((DIM, BE), lambda i: (0, i)),
        ],
        out_specs=[
            pl.BlockSpec((K, BN, OUT_F), lambda i: (0, i, 0)),
            pl.BlockSpec((K, BE), lambda i: (0, i)),
        ],
        out_shape=[
            jax.ShapeDtypeStruct((K, N, OUT_F), jnp.bfloat16),
            jax.ShapeDtypeStruct((K, E), jnp.float32),
        ],
    )(mu, inv_sigma, feat, W, pseudo_t)


def _sc_gather_scatter(h4, w_e, src, dst):
    mesh = plsc.VectorSubcoreMesh(core_axis_name="c", subcore_axis_name="s")

    @functools.partial(
        pl.kernel,
        mesh=mesh,
        out_type=jax.ShapeDtypeStruct((NC, N, OUT_F), jnp.float32),
        scratch_types=(
            [pltpu.VMEM((OB,), jnp.int32)] * 2
            + [pltpu.VMEM((OB,), jnp.int32)] * 2
            + [pltpu.VMEM((OB * K + LANES,), jnp.float32)] * 2
            + [pltpu.VMEM((2 * B,), jnp.int32)] * 2
            + [pltpu.VMEM((B,), jnp.int32)] * 4
            + [
                pltpu.VMEM((TAIL,), jnp.int32),
                pltpu.VMEM((TAIL,), jnp.int32),
                pltpu.VMEM((2 * TAIL,), jnp.int32),
                pltpu.VMEM((2, 2 * B, OUT_F), jnp.int32),
                pltpu.VMEM((2, B, OUT_F), jnp.float32),
                pltpu.VMEM_SHARED((N, OUT_F), jnp.float32),
            ]
            + [pltpu.SemaphoreType.DMA] * 6
        ),
    )
    def body(h4_hbm, w_hbm, src_hbm, dst_hbm, out_hbm,
             so_v0, so_v1, do_v0, do_v1, wo_v0, wo_v1, ix0, ix1,
             dr0, dr1, dr2, dr3, st_src, st_dst, st_ix, g_v, m_v, acc_sh,
             sob0, sob1, sg0, sg1, ss0, ss1):
        cid = lax.axis_index("c")
        sid = lax.axis_index("s")
        wid = sid * NC + cid
        so_vs = (so_v0, so_v1)
        do_vs = (do_v0, do_v1)
        wo_vs = (wo_v0, wo_v1)
        ixs = (ix0, ix1)
        drs = (dr0, dr1, dr2, dr3)
        sems_o = (sob0, sob1)
        sems_g = (sg0, sg1)
        sems_s = (ss0, ss1)
        ebase0 = wid * EPW

        def fire_outer(slot, t):
            base = ebase0 + t * OB
            pltpu.async_copy(src_hbm.at[pl.ds(base, OB)], so_vs[slot],
                             sems_o[slot])
            pltpu.async_copy(dst_hbm.at[pl.ds(base, OB)], do_vs[slot],
                             sems_o[slot])
            pltpu.async_copy(w_hbm.at[pl.ds(base * K, OB * K)],
                             wo_vs[slot].at[pl.ds(0, OB * K)], sems_o[slot])

        def wait_outer(slot):
            pltpu.make_async_copy(src_hbm.at[pl.ds(0, OB)], so_vs[slot],
                                  sems_o[slot]).wait()
            pltpu.make_async_copy(dst_hbm.at[pl.ds(0, OB)], do_vs[slot],
                                  sems_o[slot]).wait()
            pltpu.make_async_copy(w_hbm.at[pl.ds(0, OB * K)],
                                  wo_vs[slot].at[pl.ds(0, OB * K)],
                                  sems_o[slot]).wait()

        def build_ix(islot, oslot, off):
            for grp in range(B // LANES):
                s = so_vs[oslot][pl.ds(off + grp * LANES, LANES)]
                for kk in range(2):
                    v = s if kk == 0 else s + (kk * N)
                    ixs[islot][pl.ds(kk * B + grp * LANES, LANES)] = v

        def fire_gather(j):
            pltpu.async_copy(h4_hbm.at[ixs[j]], g_v.at[j], sems_g[j])

        def wait_gather(j):
            pltpu.make_async_copy(h4_hbm.at[ixs[j]], g_v.at[j],
                                  sems_g[j]).wait()

        def build_dr(rslot, oslot, off):
            for grp in range(B // LANES):
                drs[rslot][pl.ds(grp * LANES, LANES)] = (
                    do_vs[oslot][pl.ds(off + grp * LANES, LANES)])

        def fire_scatter(j, rslot):
            pltpu.async_copy(m_v.at[j], acc_sh.at[drs[rslot]], sems_s[j],
                             add=True)

        def wait_scatter(j, rslot):
            pltpu.make_async_copy(m_v.at[j], acc_sh.at[drs[rslot]],
                                  sems_s[j]).wait()

        def combine_edges(w_ref, woff, gplane, gstride, mplane, nedge):
            himask = jnp.full((LANES,), -65536, jnp.int32)

            def edge(i, carry):
                wvec = w_ref[pl.ds(woff + i * K, LANES)]
                wks = [jnp.full((LANES,), wvec[k], jnp.float32)
                       for k in range(K)]
                for grp in range(OUT_F // (2 * LANES)):
                    ae = None
                    ao = None
                    for k in range(K):
                        v = gplane[gstride * (k // 2) + i,
                                   pl.ds((k % 2) * (OUT_F // 2)
                                         + LANES * grp, LANES)]
                        fe = lax.bitcast_convert_type(v << 16, jnp.float32)
                        fo = lax.bitcast_convert_type(v & himask, jnp.float32)
                        te = wks[k] * fe
                        to = wks[k] * fo
                        ae = te if ae is None else ae + te
                        ao = to if ao is None else ao + to
                    mplane[i, pl.ds(2 * LANES * grp, LANES)] = ae
                    mplane[i, pl.ds(2 * LANES * grp + LANES, LANES)] = ao
                return carry
            lax.fori_loop(0, nedge, edge, 0)

        zero = jnp.zeros((LANES,), jnp.float32)

        def zrow(r, c2):
            for c in range(OUT_F // LANES):
                m_v[0, r, pl.ds(c * LANES, LANES)] = zero
            return c2

        lax.fori_loop(0, B, zrow, 0)

        def zero_rows(start, cnt):
            for q in range(cnt // B):
                pltpu.async_copy(m_v.at[0], acc_sh.at[pl.ds(start + q * B, B)],
                                 sg0)
            rem = cnt - (cnt // B) * B
            if rem:
                pltpu.async_copy(m_v.at[0, pl.ds(0, rem)],
                                 acc_sh.at[pl.ds(start + (cnt // B) * B, rem)],
                                 sg0)
            for q in range(cnt // B):
                pltpu.make_async_copy(
                    m_v.at[0], acc_sh.at[pl.ds(start + q * B, B)], sg0).wait()
            if rem:
                pltpu.make_async_copy(
                    m_v.at[0, pl.ds(0, rem)],
                    acc_sh.at[pl.ds(start + (cnt // B) * B, rem)], sg0).wait()

        @pl.when(sid < NS - 1)
        def _():
            zero_rows(sid * ROWS8, ROWS8)

        @pl.when(sid == NS - 1)
        def _():
            zero_rows((NS - 1) * ROWS8, ROWS_LAST)

        plsc.subcore_barrier()

        fire_outer(0, 0)
        wait_outer(0)
        build_ix(0, 0, 0)
        fire_gather(0)

        def emit_chunk(t, tpar, bb):
            c = t * UNROLL + bb
            j = bb % 2
            o = 1 - j
            npar = 1 - tpar

            @pl.when(c < NCHUNK - 1)
            def _():
                if bb == 3:
                    wait_outer(npar)
                    build_ix(o, npar, 0)
                else:
                    build_ix(o, tpar, (bb + 1) * B)
                fire_gather(o)

            if bb < 2:
                @pl.when(t > 0)
                def _():
                    wait_scatter(j, (bb + 2) % 4)
            else:
                wait_scatter(j, (bb + 2) % 4)

            wait_gather(j)
            combine_edges(wo_vs[tpar], bb * B * K, g_v.at[j], B, m_v.at[j], B)
            build_dr(bb, tpar, bb * B)
            fire_scatter(j, bb)

        def outer(tt, carry):
            for tpar in range(2):
                t = tt * 2 + tpar

                @pl.when(t + 1 < T_ITERS)
                def _():
                    fire_outer(1 - tpar, t + 1)

                for bb in range(UNROLL):
                    emit_chunk(t, tpar, bb)
            return carry

        lax.fori_loop(0, T_ITERS // 2, outer, 0)

        wait_scatter(0, 2)
        wait_scatter(1, 3)

        tbase = ebase0 + NCHUNK * B
        pltpu.sync_copy(src_hbm.at[pl.ds(tbase, TAIL)], st_src)
        pltpu.sync_copy(dst_hbm.at[pl.ds(tbase, TAIL)], st_dst)
        pltpu.sync_copy(w_hbm.at[pl.ds(tbase * K, TAIL * K)],
                        wo_v0.at[pl.ds(0, TAIL * K)])
        s = st_src[...]
        for kk in range(2):
            st_ix[pl.ds(kk * TAIL, LANES)] = s if kk == 0 else s + (kk * N)
        pltpu.async_copy(h4_hbm.at[st_ix], g_v.at[0, pl.ds(0, 2 * TAIL)],
                         sg0).wait()
        combine_edges(wo_v0, 0, g_v.at[0], TAIL, m_v.at[0], TAIL)
        pltpu.sync_copy(m_v.at[0, pl.ds(0, TAIL)], acc_sh.at[st_dst],
                        add=True)

        plsc.subcore_barrier()

        @pl.when(sid < NS - 1)
        def _():
            pltpu.sync_copy(acc_sh.at[pl.ds(sid * ROWS8, ROWS8)],
                            out_hbm.at[cid, pl.ds(sid * ROWS8, ROWS8)])

        @pl.when(sid == NS - 1)
        def _():
            pltpu.sync_copy(acc_sh.at[pl.ds((NS - 1) * ROWS8, ROWS_LAST)],
                            out_hbm.at[cid, pl.ds((NS - 1) * ROWS8, ROWS_LAST)])

    return body(h4, w_e, src, dst)


def _post_kernel(p_ref, b_ref, o_ref):
    o_ref[...] = p_ref[0] + p_ref[1] + b_ref[...]


def _tc_post(partials, bias2d):
    return pl.pallas_call(
        _post_kernel,
        grid=(N // BN2,),
        in_specs=[
            pl.BlockSpec((NC, BN2, OUT_F), lambda i: (0, i, 0)),
            pl.BlockSpec((1, OUT_F), lambda i: (0, 0)),
        ],
        out_specs=pl.BlockSpec((BN2, OUT_F), lambda i: (i, 0)),
        out_shape=jax.ShapeDtypeStruct((N, OUT_F), jnp.float32),
    )(partials, bias2d)


def kernel(feat, pseudo, edge_index, W, mu, inv_sigma, bias):
    pseudo_t = pseudo.T
    h4, wt = _tc_pre(feat, W, pseudo_t, mu, inv_sigma)
    t4 = lax.bitcast_convert_type(
        h4.reshape(K, N, OUT_F // 2, 2), jnp.int32)
    t4 = t4.reshape(2, 2, N, OUT_F // 2).transpose(0, 2, 1, 3)
    t4 = t4.reshape(2 * N, OUT_F)
    w_e = wt.T.reshape(E * K)
    src = edge_index[0]
    dst = edge_index[1]
    partials = _sc_gather_scatter(t4, w_e, src, dst)

    bias_p = bias[jnp.asarray(_PERM)]
    out_p = _tc_post(partials, bias_p.reshape(1, OUT_F))
    return out_p[:, jnp.asarray(_INV_PERM)]


# speedup vs baseline: 1.1528x; 1.1528x over previous
import functools

import jax
import jax.numpy as jnp
import numpy as np
from jax import lax
from jax.experimental import pallas as pl
from jax.experimental.pallas import tpu as pltpu
from jax.experimental.pallas import tpu_sc as plsc

N = 10000
E = 320000
IN_F = 128
OUT_F = 128
K = 4
DIM = 4

NC = 2
NS = 16
NW = NC * NS
EPW = E // NW
B = 32
TAIL = EPW % B
NCHUNK = EPW // B
UNROLL = 4
T_ITERS = NCHUNK // UNROLL
LANES = 16
OB = UNROLL * B

BN = 400
BE = E // (N // BN)
BN2 = 400

ROWS8 = 632
ROWS_LAST = N - ROWS8 * (NS - 1)

_PERM = np.empty(OUT_F, np.int32)
for _g in range(OUT_F // 32):
    for _l in range(LANES):
        _PERM[32 * _g + _l] = 16 * _g + _l
        _PERM[32 * _g + LANES + _l] = 64 + 16 * _g + _l
_INV_PERM = np.empty(OUT_F, np.int32)
_INV_PERM[_PERM] = np.arange(OUT_F, dtype=np.int32)


def _pre_kernel(mu_ref, isig_ref, feat_ref, w_ref, pt_ref, tbl_ref, wt_ref):
    hfull = lax.dot_general(
        feat_ref[...], w_ref[...], (((1,), (1,)), ((), ())),
        preferred_element_type=jnp.float32)
    hb = hfull.astype(jnp.bfloat16)

    def pack_k(k):
        hk = hb[:, k * OUT_F:(k + 1) * OUT_F]
        lo = lax.bitcast_convert_type(
            hk[:, :OUT_F // 2], jnp.uint16).astype(jnp.uint32)
        hi = lax.bitcast_convert_type(
            hk[:, OUT_F // 2:], jnp.uint16).astype(jnp.uint32)
        return lax.bitcast_convert_type(lo | (hi << 16), jnp.float32)

    for kk in range(2):
        tbl_ref[kk] = jnp.concatenate(
            [pack_k(2 * kk), pack_k(2 * kk + 1)], axis=1)
    for k in range(K):
        acc = None
        for d in range(DIM):
            p = pt_ref[d:d + 1, :]
            diff = p - mu_ref[k, d]
            t = (isig_ref[k, d] * isig_ref[k, d]) * diff * diff
            acc = t if acc is None else acc + t
        wt_ref[k:k + 1, :] = jnp.exp(-0.5 * acc)


def _tc_pre(feat, W, pseudo_t, mu, inv_sigma):
    return pl.pallas_call(
        _pre_kernel,
        grid=(N // BN,),
        in_specs=[
            pl.BlockSpec(memory_space=pltpu.SMEM),
            pl.BlockSpec(memory_space=pltpu.SMEM),
            pl.BlockSpec((BN, IN_F), lambda i: (i, 0)),
            pl.BlockSpec((K * OUT_F, IN_F), lambda i: (0, 0)),
            pl.BlockSpec((DIM, BE), lambda i: (0, i)),
        ],
        out_specs=[
            pl.BlockSpec((2, BN, OUT_F), lambda i: (0, i, 0)),
            pl.BlockSpec((K, BE), lambda i: (0, i)),
        ],
        out_shape=[
            jax.ShapeDtypeStruct((2, N, OUT_F), jnp.float32),
            jax.ShapeDtypeStruct((K, E), jnp.float32),
        ],
    )(mu, inv_sigma, feat, W, pseudo_t)


def _sc_gather_scatter(h4, w_e, src, dst):
    mesh = plsc.VectorSubcoreMesh(core_axis_name="c", subcore_axis_name="s")

    @functools.partial(
        pl.kernel,
        mesh=mesh,
        out_type=jax.ShapeDtypeStruct((NC, N, OUT_F), jnp.float32),
        scratch_types=(
            [pltpu.VMEM((OB,), jnp.int32)] * 2
            + [pltpu.VMEM((OB,), jnp.int32)] * 2
            + [pltpu.VMEM((OB * K + LANES,), jnp.float32)] * 2
            + [pltpu.VMEM((2 * B,), jnp.int32)] * 2
            + [pltpu.VMEM((B,), jnp.int32)] * 4
            + [
                pltpu.VMEM((TAIL,), jnp.int32),
                pltpu.VMEM((TAIL,), jnp.int32),
                pltpu.VMEM((2 * TAIL,), jnp.int32),
                pltpu.VMEM((2, 2 * B, OUT_F), jnp.int32),
                pltpu.VMEM((2, B, OUT_F), jnp.float32),
                pltpu.VMEM_SHARED((N, OUT_F), jnp.float32),
            ]
            + [pltpu.SemaphoreType.DMA] * 6
        ),
    )
    def body(h4_hbm, w_hbm, src_hbm, dst_hbm, out_hbm,
             so_v0, so_v1, do_v0, do_v1, wo_v0, wo_v1, ix0, ix1,
             dr0, dr1, dr2, dr3, st_src, st_dst, st_ix, g_v, m_v, acc_sh,
             sob0, sob1, sg0, sg1, ss0, ss1):
        cid = lax.axis_index("c")
        sid = lax.axis_index("s")
        wid = sid * NC + cid
        so_vs = (so_v0, so_v1)
        do_vs = (do_v0, do_v1)
        wo_vs = (wo_v0, wo_v1)
        ixs = (ix0, ix1)
        drs = (dr0, dr1, dr2, dr3)
        sems_o = (sob0, sob1)
        sems_g = (sg0, sg1)
        sems_s = (ss0, ss1)
        ebase0 = wid * EPW

        def fire_outer(slot, t):
            base = ebase0 + t * OB
            pltpu.async_copy(src_hbm.at[pl.ds(base, OB)], so_vs[slot],
                             sems_o[slot])
            pltpu.async_copy(dst_hbm.at[pl.ds(base, OB)], do_vs[slot],
                             sems_o[slot])
            pltpu.async_copy(w_hbm.at[pl.ds(base * K, OB * K)],
                             wo_vs[slot].at[pl.ds(0, OB * K)], sems_o[slot])

        def wait_outer(slot):
            pltpu.make_async_copy(src_hbm.at[pl.ds(0, OB)], so_vs[slot],
                                  sems_o[slot]).wait()
            pltpu.make_async_copy(dst_hbm.at[pl.ds(0, OB)], do_vs[slot],
                                  sems_o[slot]).wait()
            pltpu.make_async_copy(w_hbm.at[pl.ds(0, OB * K)],
                                  wo_vs[slot].at[pl.ds(0, OB * K)],
                                  sems_o[slot]).wait()

        def build_ix(islot, oslot, off):
            for grp in range(B // LANES):
                s = so_vs[oslot][pl.ds(off + grp * LANES, LANES)]
                for kk in range(2):
                    v = s if kk == 0 else s + (kk * N)
                    ixs[islot][pl.ds(kk * B + grp * LANES, LANES)] = v

        def fire_gather(j):
            pltpu.async_copy(h4_hbm.at[ixs[j]], g_v.at[j], sems_g[j])

        def wait_gather(j):
            pltpu.make_async_copy(h4_hbm.at[ixs[j]], g_v.at[j],
                                  sems_g[j]).wait()

        def build_dr(rslot, oslot, off):
            for grp in range(B // LANES):
                drs[rslot][pl.ds(grp * LANES, LANES)] = (
                    do_vs[oslot][pl.ds(off + grp * LANES, LANES)])

        def fire_scatter(j, rslot):
            pltpu.async_copy(m_v.at[j], acc_sh.at[drs[rslot]], sems_s[j],
                             add=True)

        def wait_scatter(j, rslot):
            pltpu.make_async_copy(m_v.at[j], acc_sh.at[drs[rslot]],
                                  sems_s[j]).wait()

        def combine_edges(w_ref, woff, gplane, gstride, mplane, nedge):
            himask = jnp.full((LANES,), -65536, jnp.int32)

            def edge(i, carry):
                wvec = w_ref[pl.ds(woff + i * K, LANES)]
                wks = [jnp.full((LANES,), wvec[k], jnp.float32)
                       for k in range(K)]
                for grp in range(OUT_F // (2 * LANES)):
                    ae = None
                    ao = None
                    for k in range(K):
                        v = gplane[gstride * (k // 2) + i,
                                   pl.ds((k % 2) * (OUT_F // 2)
                                         + LANES * grp, LANES)]
                        fe = lax.bitcast_convert_type(v << 16, jnp.float32)
                        fo = lax.bitcast_convert_type(v & himask, jnp.float32)
                        te = wks[k] * fe
                        to = wks[k] * fo
                        ae = te if ae is None else ae + te
                        ao = to if ao is None else ao + to
                    mplane[i, pl.ds(2 * LANES * grp, LANES)] = ae
                    mplane[i, pl.ds(2 * LANES * grp + LANES, LANES)] = ao
                return carry
            lax.fori_loop(0, nedge, edge, 0)

        zero = jnp.zeros((LANES,), jnp.float32)

        def zrow(r, c2):
            for c in range(OUT_F // LANES):
                m_v[0, r, pl.ds(c * LANES, LANES)] = zero
            return c2

        lax.fori_loop(0, B, zrow, 0)

        def zero_rows(start, cnt):
            for q in range(cnt // B):
                pltpu.async_copy(m_v.at[0], acc_sh.at[pl.ds(start + q * B, B)],
                                 sg0)
            rem = cnt - (cnt // B) * B
            if rem:
                pltpu.async_copy(m_v.at[0, pl.ds(0, rem)],
                                 acc_sh.at[pl.ds(start + (cnt // B) * B, rem)],
                                 sg0)
            for q in range(cnt // B):
                pltpu.make_async_copy(
                    m_v.at[0], acc_sh.at[pl.ds(start + q * B, B)], sg0).wait()
            if rem:
                pltpu.make_async_copy(
                    m_v.at[0, pl.ds(0, rem)],
                    acc_sh.at[pl.ds(start + (cnt // B) * B, rem)], sg0).wait()

        @pl.when(sid < NS - 1)
        def _():
            zero_rows(sid * ROWS8, ROWS8)

        @pl.when(sid == NS - 1)
        def _():
            zero_rows((NS - 1) * ROWS8, ROWS_LAST)

        plsc.subcore_barrier()

        fire_outer(0, 0)
        wait_outer(0)
        build_ix(0, 0, 0)
        fire_gather(0)

        def emit_chunk(t, tpar, bb):
            c = t * UNROLL + bb
            j = bb % 2
            o = 1 - j
            npar = 1 - tpar

            @pl.when(c < NCHUNK - 1)
            def _():
                if bb == 3:
                    wait_outer(npar)
                    build_ix(o, npar, 0)
                else:
                    build_ix(o, tpar, (bb + 1) * B)
                fire_gather(o)

            if bb < 2:
                @pl.when(t > 0)
                def _():
                    wait_scatter(j, (bb + 2) % 4)
            else:
                wait_scatter(j, (bb + 2) % 4)

            wait_gather(j)
            combine_edges(wo_vs[tpar], bb * B * K, g_v.at[j], B, m_v.at[j], B)
            build_dr(bb, tpar, bb * B)
            fire_scatter(j, bb)

        def outer(tt, carry):
            for tpar in range(2):
                t = tt * 2 + tpar

                @pl.when(t + 1 < T_ITERS)
                def _():
                    fire_outer(1 - tpar, t + 1)

                for bb in range(UNROLL):
                    emit_chunk(t, tpar, bb)
            return carry

        lax.fori_loop(0, T_ITERS // 2, outer, 0)

        wait_scatter(0, 2)
        wait_scatter(1, 3)

        tbase = ebase0 + NCHUNK * B
        pltpu.sync_copy(src_hbm.at[pl.ds(tbase, TAIL)], st_src)
        pltpu.sync_copy(dst_hbm.at[pl.ds(tbase, TAIL)], st_dst)
        pltpu.sync_copy(w_hbm.at[pl.ds(tbase * K, TAIL * K)],
                        wo_v0.at[pl.ds(0, TAIL * K)])
        s = st_src[...]
        for kk in range(2):
            st_ix[pl.ds(kk * TAIL, LANES)] = s if kk == 0 else s + (kk * N)
        pltpu.async_copy(h4_hbm.at[st_ix], g_v.at[0, pl.ds(0, 2 * TAIL)],
                         sg0).wait()
        combine_edges(wo_v0, 0, g_v.at[0], TAIL, m_v.at[0], TAIL)
        pltpu.sync_copy(m_v.at[0, pl.ds(0, TAIL)], acc_sh.at[st_dst],
                        add=True)

        plsc.subcore_barrier()

        @pl.when(sid < NS - 1)
        def _():
            pltpu.sync_copy(acc_sh.at[pl.ds(sid * ROWS8, ROWS8)],
                            out_hbm.at[cid, pl.ds(sid * ROWS8, ROWS8)])

        @pl.when(sid == NS - 1)
        def _():
            pltpu.sync_copy(acc_sh.at[pl.ds((NS - 1) * ROWS8, ROWS_LAST)],
                            out_hbm.at[cid, pl.ds((NS - 1) * ROWS8, ROWS_LAST)])

    return body(h4, w_e, src, dst)


def _post_kernel(p_ref, b_ref, o_ref):
    o_ref[...] = p_ref[0] + p_ref[1] + b_ref[...]


def _tc_post(partials, bias2d):
    return pl.pallas_call(
        _post_kernel,
        grid=(N // BN2,),
        in_specs=[
            pl.BlockSpec((NC, BN2, OUT_F), lambda i: (0, i, 0)),
            pl.BlockSpec((1, OUT_F), lambda i: (0, 0)),
        ],
        out_specs=pl.BlockSpec((BN2, OUT_F), lambda i: (i, 0)),
        out_shape=jax.ShapeDtypeStruct((N, OUT_F), jnp.float32),
    )(partials, bias2d)


def kernel(feat, pseudo, edge_index, W, mu, inv_sigma, bias):
    pseudo_t = pseudo.T
    tbl, wt = _tc_pre(feat, W, pseudo_t, mu, inv_sigma)
    t4 = lax.bitcast_convert_type(tbl, jnp.int32).reshape(2 * N, OUT_F)
    w_e = wt.T.reshape(E * K)
    src = edge_index[0]
    dst = edge_index[1]
    partials = _sc_gather_scatter(t4, w_e, src, dst)

    bias_p = bias[jnp.asarray(_PERM)]
    out_p = _tc_post(partials, bias_p.reshape(1, OUT_F))
    return out_p[:, jnp.asarray(_INV_PERM)]


# speedup vs baseline: 1.2933x; 1.1218x over previous
import functools

import jax
import jax.numpy as jnp
import numpy as np
from jax import lax
from jax.experimental import pallas as pl
from jax.experimental.pallas import tpu as pltpu
from jax.experimental.pallas import tpu_sc as plsc

N = 10000
E = 320000
IN_F = 128
OUT_F = 128
K = 4
DIM = 4

NC = 2
NS = 16
NW = NC * NS
EPW = E // NW
B = 32
TAIL = EPW % B
NCHUNK = EPW // B
UNROLL = 4
T_ITERS = NCHUNK // UNROLL
LANES = 16
OB = UNROLL * B

BN = 400
BE = E // (N // BN)
BN2 = 400

ROWS8 = 632
ROWS_LAST = N - ROWS8 * (NS - 1)

_PERM = np.empty(OUT_F, np.int32)
for _g in range(OUT_F // 32):
    for _l in range(LANES):
        _PERM[32 * _g + _l] = 16 * _g + _l
        _PERM[32 * _g + LANES + _l] = 64 + 16 * _g + _l
_INV_PERM = np.empty(OUT_F, np.int32)
_INV_PERM[_PERM] = np.arange(OUT_F, dtype=np.int32)


def _pre_kernel(mu_ref, isig_ref, feat_ref, w_ref, pt_ref, tbl_ref, wt_ref):
    hfull = lax.dot_general(
        feat_ref[...], w_ref[...], (((1,), (1,)), ((), ())),
        preferred_element_type=jnp.float32)
    hb = hfull.astype(jnp.bfloat16)

    def pack_k(k):
        hk = hb[:, k * OUT_F:(k + 1) * OUT_F]
        lo = lax.bitcast_convert_type(
            hk[:, :OUT_F // 2], jnp.uint16).astype(jnp.uint32)
        hi = lax.bitcast_convert_type(
            hk[:, OUT_F // 2:], jnp.uint16).astype(jnp.uint32)
        return lax.bitcast_convert_type(lo | (hi << 16), jnp.float32)

    for kk in range(2):
        tbl_ref[kk] = jnp.concatenate(
            [pack_k(2 * kk), pack_k(2 * kk + 1)], axis=1)
    for k in range(K):
        acc = None
        for d in range(DIM):
            p = pt_ref[d:d + 1, :]
            diff = p - mu_ref[k, d]
            t = (isig_ref[k, d] * isig_ref[k, d]) * diff * diff
            acc = t if acc is None else acc + t
        wt_ref[k:k + 1, :] = jnp.exp(-0.5 * acc)


def _tc_pre(feat, W, pseudo_t, mu, inv_sigma):
    return pl.pallas_call(
        _pre_kernel,
        grid=(N // BN,),
        in_specs=[
            pl.BlockSpec(memory_space=pltpu.SMEM),
            pl.BlockSpec(memory_space=pltpu.SMEM),
            pl.BlockSpec((BN, IN_F), lambda i: (i, 0)),
            pl.BlockSpec((K * OUT_F, IN_F), lambda i: (0, 0)),
            pl.BlockSpec((DIM, BE), lambda i: (0, i)),
        ],
        out_specs=[
            pl.BlockSpec((2, BN, OUT_F), lambda i: (0, i, 0)),
            pl.BlockSpec((K, BE), lambda i: (0, i)),
        ],
        out_shape=[
            jax.ShapeDtypeStruct((2, N, OUT_F), jnp.float32),
            jax.ShapeDtypeStruct((K, E), jnp.float32),
        ],
    )(mu, inv_sigma, feat, W, pseudo_t)


def _sc_gather_scatter(h4, w_e, src, dst):
    mesh = plsc.VectorSubcoreMesh(core_axis_name="c", subcore_axis_name="s")

    @functools.partial(
        pl.kernel,
        mesh=mesh,
        out_type=jax.ShapeDtypeStruct((NC, N, OUT_F), jnp.float32),
        scratch_types=(
            [pltpu.VMEM((OB,), jnp.int32)] * 2
            + [pltpu.VMEM((OB,), jnp.int32)] * 2
            + [pltpu.VMEM((OB * K + LANES,), jnp.float32)] * 2
            + [pltpu.VMEM((2 * B,), jnp.int32)] * 4
            + [pltpu.VMEM((B,), jnp.int32)] * 4
            + [
                pltpu.VMEM((TAIL,), jnp.int32),
                pltpu.VMEM((TAIL,), jnp.int32),
                pltpu.VMEM((2 * TAIL,), jnp.int32),
                pltpu.VMEM((4, 2 * B, OUT_F), jnp.int32),
                pltpu.VMEM((2, B, OUT_F), jnp.float32),
                pltpu.VMEM_SHARED((N, OUT_F), jnp.float32),
            ]
            + [pltpu.SemaphoreType.DMA] * 8
        ),
    )
    def body(h4_hbm, w_hbm, src_hbm, dst_hbm, out_hbm,
             so_v0, so_v1, do_v0, do_v1, wo_v0, wo_v1, ix0, ix1, ix2, ix3,
             dr0, dr1, dr2, dr3, st_src, st_dst, st_ix, g_v, m_v, acc_sh,
             sob0, sob1, sg0, sg1, sg2, sg3, ss0, ss1):
        cid = lax.axis_index("c")
        sid = lax.axis_index("s")
        wid = sid * NC + cid
        so_vs = (so_v0, so_v1)
        do_vs = (do_v0, do_v1)
        wo_vs = (wo_v0, wo_v1)
        ixs = (ix0, ix1, ix2, ix3)
        drs = (dr0, dr1, dr2, dr3)
        sems_o = (sob0, sob1)
        sems_g = (sg0, sg1, sg2, sg3)
        sems_s = (ss0, ss1)
        ebase0 = wid * EPW

        def fire_outer(slot, t):
            base = ebase0 + t * OB
            pltpu.async_copy(src_hbm.at[pl.ds(base, OB)], so_vs[slot],
                             sems_o[slot])
            pltpu.async_copy(dst_hbm.at[pl.ds(base, OB)], do_vs[slot],
                             sems_o[slot])
            pltpu.async_copy(w_hbm.at[pl.ds(base * K, OB * K)],
                             wo_vs[slot].at[pl.ds(0, OB * K)], sems_o[slot])

        def wait_outer(slot):
            pltpu.make_async_copy(src_hbm.at[pl.ds(0, OB)], so_vs[slot],
                                  sems_o[slot]).wait()
            pltpu.make_async_copy(dst_hbm.at[pl.ds(0, OB)], do_vs[slot],
                                  sems_o[slot]).wait()
            pltpu.make_async_copy(w_hbm.at[pl.ds(0, OB * K)],
                                  wo_vs[slot].at[pl.ds(0, OB * K)],
                                  sems_o[slot]).wait()

        def build_ix(islot, oslot, off):
            for grp in range(B // LANES):
                s = so_vs[oslot][pl.ds(off + grp * LANES, LANES)]
                for kk in range(2):
                    v = s if kk == 0 else s + (kk * N)
                    ixs[islot][pl.ds(kk * B + grp * LANES, LANES)] = v

        def fire_gather(slot):
            pltpu.async_copy(h4_hbm.at[ixs[slot]], g_v.at[slot],
                             sems_g[slot])

        def wait_gather(slot):
            pltpu.make_async_copy(h4_hbm.at[ixs[slot]], g_v.at[slot],
                                  sems_g[slot]).wait()

        def build_dr(rslot, oslot, off):
            for grp in range(B // LANES):
                drs[rslot][pl.ds(grp * LANES, LANES)] = (
                    do_vs[oslot][pl.ds(off + grp * LANES, LANES)])

        def fire_scatter(j, rslot):
            pltpu.async_copy(m_v.at[j], acc_sh.at[drs[rslot]], sems_s[j],
                             add=True)

        def wait_scatter(j, rslot):
            pltpu.make_async_copy(m_v.at[j], acc_sh.at[drs[rslot]],
                                  sems_s[j]).wait()

        def combine_edges(w_ref, woff, gplane, gstride, mplane, nedge):
            himask = jnp.full((LANES,), -65536, jnp.int32)

            def edge(i, carry):
                wvec = w_ref[pl.ds(woff + i * K, LANES)]
                wks = [jnp.full((LANES,), wvec[k], jnp.float32)
                       for k in range(K)]
                for grp in range(OUT_F // (2 * LANES)):
                    ae = None
                    ao = None
                    for k in range(K):
                        v = gplane[gstride * (k // 2) + i,
                                   pl.ds((k % 2) * (OUT_F // 2)
                                         + LANES * grp, LANES)]
                        fe = lax.bitcast_convert_type(v << 16, jnp.float32)
                        fo = lax.bitcast_convert_type(v & himask, jnp.float32)
                        te = wks[k] * fe
                        to = wks[k] * fo
                        ae = te if ae is None else ae + te
                        ao = to if ao is None else ao + to
                    mplane[i, pl.ds(2 * LANES * grp, LANES)] = ae
                    mplane[i, pl.ds(2 * LANES * grp + LANES, LANES)] = ao
                return carry
            lax.fori_loop(0, nedge, edge, 0)

        zero = jnp.zeros((LANES,), jnp.float32)

        def zrow(r, c2):
            for c in range(OUT_F // LANES):
                m_v[0, r, pl.ds(c * LANES, LANES)] = zero
            return c2

        lax.fori_loop(0, B, zrow, 0)

        def zero_rows(start, cnt):
            for q in range(cnt // B):
                pltpu.async_copy(m_v.at[0], acc_sh.at[pl.ds(start + q * B, B)],
                                 sg0)
            rem = cnt - (cnt // B) * B
            if rem:
                pltpu.async_copy(m_v.at[0, pl.ds(0, rem)],
                                 acc_sh.at[pl.ds(start + (cnt // B) * B, rem)],
                                 sg0)
            for q in range(cnt // B):
                pltpu.make_async_copy(
                    m_v.at[0], acc_sh.at[pl.ds(start + q * B, B)], sg0).wait()
            if rem:
                pltpu.make_async_copy(
                    m_v.at[0, pl.ds(0, rem)],
                    acc_sh.at[pl.ds(start + (cnt // B) * B, rem)], sg0).wait()

        @pl.when(sid < NS - 1)
        def _():
            zero_rows(sid * ROWS8, ROWS8)

        @pl.when(sid == NS - 1)
        def _():
            zero_rows((NS - 1) * ROWS8, ROWS_LAST)

        plsc.subcore_barrier()

        fire_outer(0, 0)
        wait_outer(0)
        build_ix(0, 0, 0)
        fire_gather(0)
        build_ix(1, 0, B)
        fire_gather(1)

        def emit_chunk(t, tpar, bb):
            c = t * UNROLL + bb
            j = bb % 2
            s2 = (bb + 2) % 4
            npar = 1 - tpar

            @pl.when(c < NCHUNK - 2)
            def _():
                if bb >= 2:
                    if bb == 2:
                        wait_outer(npar)
                    build_ix(s2, npar, (bb - 2) * B)
                else:
                    build_ix(s2, tpar, (bb + 2) * B)
                fire_gather(s2)

            if bb < 2:
                @pl.when(t > 0)
                def _():
                    wait_scatter(j, s2)
            else:
                wait_scatter(j, s2)

            wait_gather(bb)
            combine_edges(wo_vs[tpar], bb * B * K, g_v.at[bb], B, m_v.at[j], B)
            build_dr(bb, tpar, bb * B)
            fire_scatter(j, bb)

        def outer(tt, carry):
            for tpar in range(2):
                t = tt * 2 + tpar

                @pl.when(t + 1 < T_ITERS)
                def _():
                    fire_outer(1 - tpar, t + 1)

                for bb in range(UNROLL):
                    emit_chunk(t, tpar, bb)
            return carry

        lax.fori_loop(0, T_ITERS // 2, outer, 0)

        wait_scatter(0, 2)
        wait_scatter(1, 3)

        tbase = ebase0 + NCHUNK * B
        pltpu.sync_copy(src_hbm.at[pl.ds(tbase, TAIL)], st_src)
        pltpu.sync_copy(dst_hbm.at[pl.ds(tbase, TAIL)], st_dst)
        pltpu.sync_copy(w_hbm.at[pl.ds(tbase * K, TAIL * K)],
                        wo_v0.at[pl.ds(0, TAIL * K)])
        s = st_src[...]
        for kk in range(2):
            st_ix[pl.ds(kk * TAIL, LANES)] = s if kk == 0 else s + (kk * N)
        pltpu.async_copy(h4_hbm.at[st_ix], g_v.at[0, pl.ds(0, 2 * TAIL)],
                         sg0).wait()
        combine_edges(wo_v0, 0, g_v.at[0], TAIL, m_v.at[0], TAIL)
        pltpu.sync_copy(m_v.at[0, pl.ds(0, TAIL)], acc_sh.at[st_dst],
                        add=True)

        plsc.subcore_barrier()

        @pl.when(sid < NS - 1)
        def _():
            pltpu.sync_copy(acc_sh.at[pl.ds(sid * ROWS8, ROWS8)],
                            out_hbm.at[cid, pl.ds(sid * ROWS8, ROWS8)])

        @pl.when(sid == NS - 1)
        def _():
            pltpu.sync_copy(acc_sh.at[pl.ds((NS - 1) * ROWS8, ROWS_LAST)],
                            out_hbm.at[cid, pl.ds((NS - 1) * ROWS8, ROWS_LAST)])

    return body(h4, w_e, src, dst)


def _post_kernel(p_ref, b_ref, o_ref):
    o_ref[...] = p_ref[0] + p_ref[1] + b_ref[...]


def _tc_post(partials, bias2d):
    return pl.pallas_call(
        _post_kernel,
        grid=(N // BN2,),
        in_specs=[
            pl.BlockSpec((NC, BN2, OUT_F), lambda i: (0, i, 0)),
            pl.BlockSpec((1, OUT_F), lambda i: (0, 0)),
        ],
        out_specs=pl.BlockSpec((BN2, OUT_F), lambda i: (i, 0)),
        out_shape=jax.ShapeDtypeStruct((N, OUT_F), jnp.float32),
    )(partials, bias2d)


def kernel(feat, pseudo, edge_index, W, mu, inv_sigma, bias):
    pseudo_t = pseudo.T
    tbl, wt = _tc_pre(feat, W, pseudo_t, mu, inv_sigma)
    t4 = lax.bitcast_convert_type(tbl, jnp.int32).reshape(2 * N, OUT_F)
    w_e = wt.T.reshape(E * K)
    src = edge_index[0]
    dst = edge_index[1]
    partials = _sc_gather_scatter(t4, w_e, src, dst)

    bias_p = bias[jnp.asarray(_PERM)]
    out_p = _tc_post(partials, bias_p.reshape(1, OUT_F))
    return out_p[:, jnp.asarray(_INV_PERM)]
